# baseline (speedup 1.0000x reference)
import functools

import jax
import jax.numpy as jnp
from jax.experimental import pallas as pl
from jax.experimental.pallas import tpu as pltpu


def _fused_kernel(x_ref, a_hbm, bg_ref, wg_ref, wd_ref, bd_ref, out_ref,
                  hall_ref, a_vmem, sems, *, PA, PB, BA, Q, B, H):
    s = pl.program_id(0)

    for j in range(PA):
        @pl.when(s == j)
        def _phase_a(j=j):
            if j < PB:
                pltpu.make_async_copy(
                    a_hbm.at[pl.ds(j * BA, BA), :],
                    a_vmem.at[pl.ds(j * BA, BA), :],
                    sems.at[j],
                ).start()
            wg = wg_ref[...].astype(jnp.bfloat16)
            hs = [
                jax.lax.dot_general(
                    x_ref[q].astype(jnp.bfloat16), wg,
                    dimension_numbers=(((0,), (0,)), ((), ())),
                    preferred_element_type=jnp.float32,
                )
                for q in range(Q)
            ]
            hall_ref[:, Q * H * j:Q * H * (j + 1)] = (
                jnp.concatenate(hs, axis=1).astype(jnp.bfloat16))

    for i in range(PB):
        @pl.when(s == PA + i)
        def _phase_b(i=i):
            pltpu.make_async_copy(
                a_hbm.at[pl.ds(i * BA, BA), :],
                a_vmem.at[pl.ds(i * BA, BA), :],
                sems.at[i],
            ).wait()
            a_blk = a_vmem[pl.ds(i * BA, BA), :].astype(jnp.bfloat16)
            g = jnp.dot(a_blk, hall_ref[...], preferred_element_type=jnp.float32)
            wd = wd_ref[...].astype(jnp.bfloat16)
            for b in range(B):
                gb = jnp.maximum(g[:, b * H:(b + 1) * H] + bg_ref[...], 0.0)
                ob = jnp.dot(gb.astype(jnp.bfloat16), wd,
                             preferred_element_type=jnp.float32)
                out_ref[b] = jnp.maximum(ob + bd_ref[...], 0.0)


def kernel(x, a, W_gcn, b_gcn, W_d, b_d):
    B, T, N = x.shape
    H = W_gcn.shape[1]
    bg = b_gcn.reshape(1, H)
    bd = b_d.reshape(1, T)

    Q = 4
    PA = B // Q
    BA = 512
    PB = N // BA

    return pl.pallas_call(
        functools.partial(_fused_kernel, PA=PA, PB=PB, BA=BA, Q=Q, B=B, H=H),
        grid=(PA + PB,),
        in_specs=[
            pl.BlockSpec((Q, T, N), lambda s: (jnp.minimum(s, PA - 1), 0, 0)),
            pl.BlockSpec(memory_space=pl.ANY),
            pl.BlockSpec((1, H), lambda s: (0, 0)),
            pl.BlockSpec((T, H), lambda s: (0, 0)),
            pl.BlockSpec((H, T), lambda s: (0, 0)),
            pl.BlockSpec((1, T), lambda s: (0, 0)),
        ],
        out_specs=pl.BlockSpec(
            (B, BA, T), lambda s: (0, jnp.maximum(s - PA, 0), 0)),
        out_shape=jax.ShapeDtypeStruct((B, N, T), jnp.float32),
        scratch_shapes=[
            pltpu.VMEM((N, B * H), jnp.bfloat16),
            pltpu.VMEM((N, N), jnp.float32),
            pltpu.SemaphoreType.DMA((PB,)),
        ],
    )(x, a, bg, W_gcn, W_d, bd)


# speedup vs baseline: 1.0251x; 1.0251x over previous
import functools

import jax
import jax.numpy as jnp
from jax.experimental import pallas as pl
from jax.experimental.pallas import tpu as pltpu


def _fused_kernel(x_ref, a_ref, bg_ref, wg_ref, wd_ref, bd_ref, out_ref,
                  g_ref, *, NC, NK, BW, B, H):
    s = pl.program_id(0)

    for k in range(NC):
        @pl.when(s == k)
        def _read_acc(k=k):
            wg = wg_ref[...].astype(jnp.bfloat16)
            hs = [
                jax.lax.dot_general(
                    x_ref[b].astype(jnp.bfloat16), wg,
                    dimension_numbers=(((0,), (0,)), ((), ())),
                    preferred_element_type=jnp.float32,
                )
                for b in range(B)
            ]
            panel = jnp.concatenate(hs, axis=1).astype(jnp.bfloat16)
            acc = jnp.dot(a_ref[...].astype(jnp.bfloat16), panel,
                          preferred_element_type=jnp.float32)
            if k == 0:
                g_ref[...] = acc
            else:
                g_ref[...] += acc

    for w in range(NC):
        @pl.when(s == NC + w)
        def _write(w=w):
            g = g_ref[pl.ds(w * BW, BW), :]
            wd = wd_ref[...].astype(jnp.bfloat16)
            for b in range(B):
                gb = jnp.maximum(g[:, b * H:(b + 1) * H] + bg_ref[...], 0.0)
                ob = jnp.dot(gb.astype(jnp.bfloat16), wd,
                             preferred_element_type=jnp.float32)
                out_ref[b] = jnp.maximum(ob + bd_ref[...], 0.0)


def kernel(x, a, W_gcn, b_gcn, W_d, b_d):
    B, T, N = x.shape
    H = W_gcn.shape[1]
    bg = b_gcn.reshape(1, H)
    bd = b_d.reshape(1, T)

    NC = 4
    NK = N // NC
    BW = N // NC

    return pl.pallas_call(
        functools.partial(_fused_kernel, NC=NC, NK=NK, BW=BW, B=B, H=H),
        grid=(2 * NC,),
        in_specs=[
            pl.BlockSpec((B, T, NK), lambda s: (0, 0, jnp.minimum(s, NC - 1))),
            pl.BlockSpec((N, NK), lambda s: (0, jnp.minimum(s, NC - 1))),
            pl.BlockSpec((1, H), lambda s: (0, 0)),
            pl.BlockSpec((T, H), lambda s: (0, 0)),
            pl.BlockSpec((H, T), lambda s: (0, 0)),
            pl.BlockSpec((1, T), lambda s: (0, 0)),
        ],
        out_specs=pl.BlockSpec(
            (B, BW, T), lambda s: (0, jnp.maximum(s - NC, 0), 0)),
        out_shape=jax.ShapeDtypeStruct((B, N, T), jnp.float32),
        scratch_shapes=[pltpu.VMEM((N, B * H), jnp.float32)],
    )(x, a, bg, W_gcn, W_d, bd)
